# Initial kernel scaffold; baseline (speedup 1.0000x reference)
import jax
import jax.numpy as jnp
from jax.experimental import pallas as pl


def kernel(x, edge_index, W1, b1, W2, b2):
    raise NotImplementedError("write your pallas kernel here")


# speedup vs baseline: 31.9093x; 31.9093x over previous
import dataclasses
import functools

import jax
import jax.numpy as jnp
from jax import lax
from jax.experimental import pallas as pl
from jax.experimental.pallas import tpu as pltpu
from jax.experimental.pallas import tpu_sc as plsc

NC = 2
NS = 16
NW = NC * NS
LANES = 16


def _sc_compiler_params():
    cp = pltpu.CompilerParams()
    fields = pltpu.CompilerParams.__dataclass_fields__
    if "needs_layout_passes" in fields:
        cp = dataclasses.replace(cp, needs_layout_passes=False)
    if "use_tc_tiling_on_sc" in fields:
        cp = dataclasses.replace(cp, use_tc_tiling_on_sc=False)
    return cp


def _sc_degree_hist(dst, n_nodes):
    (e_total,) = dst.shape
    epw = e_total // NW
    mesh = plsc.VectorSubcoreMesh(core_axis_name="c", subcore_axis_name="s")

    @functools.partial(
        pl.kernel,
        out_type=jax.ShapeDtypeStruct((NW, n_nodes), jnp.float32),
        mesh=mesh,
        compiler_params=_sc_compiler_params(),
        scratch_types=[
            pltpu.VMEM((epw,), jnp.int32),
            pltpu.VMEM((n_nodes,), jnp.float32),
            pltpu.SemaphoreType.DMA,
        ],
    )
    def hist_kernel(dst_hbm, out_hbm, idx_v, hist_v, sem):
        cid = lax.axis_index("c")
        sid = lax.axis_index("s")
        wid = sid * NC + cid

        zeros = jnp.zeros((LANES,), jnp.float32)

        @pl.loop(0, n_nodes // LANES)
        def _(i):
            hist_v[pl.ds(i * LANES, LANES)] = zeros

        pltpu.async_copy(dst_hbm.at[pl.ds(wid * epw, epw)], idx_v, sem).wait()

        ones = jnp.ones((LANES,), jnp.float32)

        @pl.loop(0, epw // LANES)
        def _(i):
            idx = idx_v[pl.ds(i * LANES, LANES)]
            plsc.addupdate_scatter(hist_v, [idx], ones)

        pltpu.async_copy(hist_v, out_hbm.at[wid], sem).wait()

    return hist_kernel(dst)


def _sc_edge_pass(g, src3, dst3, zeros_rows, n_pad, chunks, k):
    mesh = plsc.VectorSubcoreMesh(core_axis_name="c", subcore_axis_name="s")
    rps = n_pad // NS

    @functools.partial(
        pl.kernel,
        out_type=jax.ShapeDtypeStruct((NC, n_pad, LANES), jnp.float32),
        mesh=mesh,
        compiler_params=_sc_compiler_params(),
        scratch_types=[
            pltpu.VMEM((chunks, k), jnp.int32),
            pltpu.VMEM((chunks, k), jnp.int32),
            pltpu.VMEM((k, LANES), jnp.float32),
            pltpu.VMEM_SHARED((n_pad, LANES), jnp.float32),
            pltpu.SemaphoreType.DMA,
            pltpu.SemaphoreType.DMA,
        ],
    )
    def edge_kernel(g_hbm, src_hbm, dst_hbm, z_hbm, out_hbm,
                    sidx_v, didx_v, rows_v, acc_sh, sem0, sem1):
        cid = lax.axis_index("c")
        sid = lax.axis_index("s")
        wid = sid * NC + cid

        pltpu.async_copy(
            z_hbm.at[pl.ds(sid * rps, rps)],
            acc_sh.at[pl.ds(sid * rps, rps)], sem0).wait()
        pltpu.async_copy(src_hbm.at[wid], sidx_v, sem0).wait()
        pltpu.async_copy(dst_hbm.at[wid], didx_v, sem1).wait()
        plsc.subcore_barrier()

        @pl.loop(0, chunks)
        def _(j):
            pltpu.async_copy(g_hbm.at[sidx_v.at[j]], rows_v, sem0).wait()
            pltpu.async_copy(rows_v, acc_sh.at[didx_v.at[j]], sem1,
                             add=True).wait()

        plsc.subcore_barrier()
        pltpu.async_copy(
            acc_sh.at[pl.ds(sid * rps, rps)],
            out_hbm.at[cid, pl.ds(sid * rps, rps)], sem0).wait()

    return edge_kernel(g, src3, dst3, zeros_rows)


def _tc_matmul(x, w):
    def body(x_ref, w_ref, o_ref):
        o_ref[...] = jnp.dot(x_ref[...], w_ref[...],
                             preferred_element_type=jnp.float32)

    return pl.pallas_call(
        body,
        out_shape=jax.ShapeDtypeStruct((x.shape[0], w.shape[1]), jnp.float32),
    )(x, w)


def _tc_dinv(hist):
    n = hist.shape[1]

    def body(h_ref, o_ref):
        deg = jnp.sum(h_ref[...], axis=0, keepdims=True) + 1.0
        o_ref[...] = jnp.broadcast_to(lax.rsqrt(deg), (8, n))

    return pl.pallas_call(
        body,
        out_shape=jax.ShapeDtypeStruct((8, n), jnp.float32),
    )(hist)


def _tc_scale(h, dinv_col):
    def body(h_ref, d_ref, o_ref):
        o_ref[...] = h_ref[...] * d_ref[...]

    return pl.pallas_call(
        body,
        out_shape=jax.ShapeDtypeStruct(h.shape, jnp.float32),
    )(h, dinv_col)


def _tc_layer1_post(acc_a, acc_b, g1, dinv_col, b1_row):
    def body(a_ref, b_ref, g_ref, d_ref, bias_ref, o_ref):
        out1 = d_ref[...] * (a_ref[...] + b_ref[...] + g_ref[...])
        z = jnp.maximum(out1 + bias_ref[...], 0.0)
        o_ref[...] = d_ref[...] * z

    return pl.pallas_call(
        body,
        out_shape=jax.ShapeDtypeStruct(g1.shape, jnp.float32),
    )(acc_a, acc_b, g1, dinv_col, b1_row)


def _tc_layer2_post(acc_a, acc_b, g2, dinv_col, w2, b2_row):
    n = g2.shape[0]
    c = w2.shape[1]

    def body(a_ref, b_ref, g_ref, d_ref, w_ref, bias_ref, o_ref):
        agg = d_ref[...] * (a_ref[...] + b_ref[...] + g_ref[...])
        o = jnp.dot(agg, w_ref[...], preferred_element_type=jnp.float32)
        o = o + bias_ref[...]
        m = jnp.max(o, axis=1, keepdims=True)
        s = o - m
        lse = jnp.log(jnp.sum(jnp.exp(s), axis=1, keepdims=True))
        o_ref[...] = s - lse

    return pl.pallas_call(
        body,
        out_shape=jax.ShapeDtypeStruct((n, c), jnp.float32),
    )(acc_a, acc_b, g2, dinv_col, w2, b2_row)


def kernel(x, edge_index, W1, b1, W2, b2):
    n, d = x.shape
    e = edge_index.shape[1]
    h = W1.shape[1]

    k = 128
    ew = NW * k
    e_pad = ((e + ew - 1) // ew) * ew
    chunks = e_pad // ew
    n_pad = ((n + 128 - 1) // 128) * 128

    src = edge_index[0]
    dst = edge_index[1]
    pad = e_pad - e
    fill = jnp.full((pad,), n, jnp.int32)
    src3 = jnp.concatenate([src, fill]).reshape(NW, chunks, k)
    dst3 = jnp.concatenate([dst, fill]).reshape(NW, chunks, k)

    zeros_rows = jnp.zeros((n_pad, LANES), jnp.float32)

    hist = _sc_degree_hist(dst, n)
    h1 = _tc_matmul(x, W1)

    dinv8 = _tc_dinv(hist)
    dinv_col = dinv8[0].reshape(n, 1)

    g1 = _tc_scale(h1, dinv_col)
    g1_pad = jnp.concatenate([g1, jnp.zeros((n_pad - n, h), jnp.float32)])
    acc1 = _sc_edge_pass(g1_pad, src3, dst3, zeros_rows, n_pad, chunks, k)
    g2 = _tc_layer1_post(acc1[0, :n], acc1[1, :n], g1, dinv_col,
                         b1.reshape(1, h))

    g2_pad = jnp.concatenate([g2, jnp.zeros((n_pad - n, h), jnp.float32)])
    acc2 = _sc_edge_pass(g2_pad, src3, dst3, zeros_rows, n_pad, chunks, k)
    out = _tc_layer2_post(acc2[0, :n], acc2[1, :n], g2, dinv_col,
                          W2, b2.reshape(1, W2.shape[1]))
    return out


# speedup vs baseline: 53.5244x; 1.6774x over previous
import dataclasses
import functools

import jax
import jax.numpy as jnp
from jax import lax
from jax.experimental import pallas as pl
from jax.experimental.pallas import tpu as pltpu
from jax.experimental.pallas import tpu_sc as plsc

NC = 2
NS = 16
NW = NC * NS
LANES = 16


def _sc_compiler_params():
    cp = pltpu.CompilerParams()
    fields = pltpu.CompilerParams.__dataclass_fields__
    if "needs_layout_passes" in fields:
        cp = dataclasses.replace(cp, needs_layout_passes=False)
    if "use_tc_tiling_on_sc" in fields:
        cp = dataclasses.replace(cp, use_tc_tiling_on_sc=False)
    return cp


def _sc_degree_hist(dst, n_nodes):
    (e_total,) = dst.shape
    epw = e_total // NW
    mesh = plsc.VectorSubcoreMesh(core_axis_name="c", subcore_axis_name="s")

    @functools.partial(
        pl.kernel,
        out_type=jax.ShapeDtypeStruct((NW, n_nodes), jnp.float32),
        mesh=mesh,
        compiler_params=_sc_compiler_params(),
        scratch_types=[
            pltpu.VMEM((epw,), jnp.int32),
            pltpu.VMEM((n_nodes,), jnp.float32),
            pltpu.SemaphoreType.DMA,
        ],
    )
    def hist_kernel(dst_hbm, out_hbm, idx_v, hist_v, sem):
        cid = lax.axis_index("c")
        sid = lax.axis_index("s")
        wid = sid * NC + cid

        zeros = jnp.zeros((LANES,), jnp.float32)

        @pl.loop(0, n_nodes // LANES)
        def _(i):
            hist_v[pl.ds(i * LANES, LANES)] = zeros

        pltpu.async_copy(dst_hbm.at[pl.ds(wid * epw, epw)], idx_v, sem).wait()

        ones = jnp.ones((LANES,), jnp.float32)

        @pl.loop(0, epw // LANES)
        def _(i):
            idx = idx_v[pl.ds(i * LANES, LANES)]
            plsc.addupdate_scatter(hist_v, [idx], ones)

        pltpu.async_copy(hist_v, out_hbm.at[wid], sem).wait()

    return hist_kernel(dst)


_NBUF = 8
_LOOK = 4


def _sc_edge_pass(g, src3, dst3, zeros_rows, n, chunks, k):
    mesh = plsc.VectorSubcoreMesh(core_axis_name="c", subcore_axis_name="s")
    rps = n // NS

    @functools.partial(
        pl.kernel,
        out_type=jax.ShapeDtypeStruct((NC, n, LANES), jnp.float32),
        mesh=mesh,
        compiler_params=_sc_compiler_params(),
        scratch_types=[
            pltpu.VMEM((chunks, k), jnp.int32),
            pltpu.VMEM((chunks, k), jnp.int32),
            pltpu.VMEM_SHARED((n, LANES), jnp.float32),
        ] + [pltpu.VMEM((k, LANES), jnp.float32)] * _NBUF
          + [pltpu.SemaphoreType.DMA] * (2 * _NBUF),
    )
    def edge_kernel(g_hbm, src_hbm, dst_hbm, z_hbm, out_hbm,
                    sidx_v, didx_v, acc_sh, *bufs_and_sems):
        rows = bufs_and_sems[:_NBUF]
        gsem = bufs_and_sems[_NBUF:2 * _NBUF]
        ssem = bufs_and_sems[2 * _NBUF:]
        cid = lax.axis_index("c")
        sid = lax.axis_index("s")
        wid = sid * NC + cid

        def issue_gather(j, b):
            pltpu.async_copy(g_hbm.at[sidx_v.at[j]], rows[b], gsem[b])

        def wait_gather(b):
            pltpu.make_async_copy(g_hbm.at[sidx_v.at[0]], rows[b],
                                  gsem[b]).wait()

        def issue_scatter(j, b):
            pltpu.async_copy(rows[b], acc_sh.at[didx_v.at[j]], ssem[b],
                             add=True)

        def wait_scatter(b):
            pltpu.make_async_copy(rows[b], acc_sh.at[didx_v.at[0]],
                                  ssem[b]).wait()

        pltpu.async_copy(
            z_hbm.at[pl.ds(sid * rps, rps)],
            acc_sh.at[pl.ds(sid * rps, rps)], gsem[0]).wait()
        pltpu.async_copy(src_hbm.at[wid], sidx_v, gsem[0]).wait()
        pltpu.async_copy(dst_hbm.at[wid], didx_v, gsem[1]).wait()
        plsc.subcore_barrier()

        for r in range(_LOOK):
            issue_gather(r, r)

        @pl.loop(0, (chunks + _NBUF - 1) // _NBUF)
        def _(q):
            for r in range(_NBUF):
                j = q * _NBUF + r
                bg = (r + _LOOK) % _NBUF

                @pl.when(j + _LOOK < chunks)
                def _():
                    @pl.when(j >= _LOOK)
                    def _():
                        wait_scatter(bg)
                    issue_gather(j + _LOOK, bg)

                @pl.when(j < chunks)
                def _():
                    wait_gather(r)
                    issue_scatter(j, r)

        for r in range(_NBUF):
            wait_scatter(r)

        plsc.subcore_barrier()
        pltpu.async_copy(
            acc_sh.at[pl.ds(sid * rps, rps)],
            out_hbm.at[cid, pl.ds(sid * rps, rps)], gsem[0]).wait()

    return edge_kernel(g, src3, dst3, zeros_rows)


def _tc_matmul(x, w):
    def body(x_ref, w_ref, o_ref):
        o_ref[...] = jnp.dot(x_ref[...], w_ref[...],
                             preferred_element_type=jnp.float32)

    return pl.pallas_call(
        body,
        out_shape=jax.ShapeDtypeStruct((x.shape[0], w.shape[1]), jnp.float32),
    )(x, w)


def _tc_dinv(hist):
    n = hist.shape[1]

    def body(h_ref, o_ref):
        deg = jnp.sum(h_ref[...], axis=0, keepdims=True) + 1.0
        o_ref[...] = jnp.broadcast_to(lax.rsqrt(deg), (8, n))

    return pl.pallas_call(
        body,
        out_shape=jax.ShapeDtypeStruct((8, n), jnp.float32),
    )(hist)


def _tc_scale(h, dinv_col):
    def body(h_ref, d_ref, o_ref):
        o_ref[...] = h_ref[...] * d_ref[...]

    return pl.pallas_call(
        body,
        out_shape=jax.ShapeDtypeStruct(h.shape, jnp.float32),
    )(h, dinv_col)


def _tc_layer1_post(acc_a, acc_b, g1, dinv_col, b1_row):
    def body(a_ref, b_ref, g_ref, d_ref, bias_ref, o_ref):
        out1 = d_ref[...] * (a_ref[...] + b_ref[...] + g_ref[...])
        z = jnp.maximum(out1 + bias_ref[...], 0.0)
        o_ref[...] = d_ref[...] * z

    return pl.pallas_call(
        body,
        out_shape=jax.ShapeDtypeStruct(g1.shape, jnp.float32),
    )(acc_a, acc_b, g1, dinv_col, b1_row)


def _tc_layer2_post(acc_a, acc_b, g2, dinv_col, w2, b2_row):
    n = g2.shape[0]
    c = w2.shape[1]

    def body(a_ref, b_ref, g_ref, d_ref, w_ref, bias_ref, o_ref):
        agg = d_ref[...] * (a_ref[...] + b_ref[...] + g_ref[...])
        o = jnp.dot(agg, w_ref[...], preferred_element_type=jnp.float32)
        o = o + bias_ref[...]
        m = jnp.max(o, axis=1, keepdims=True)
        s = o - m
        lse = jnp.log(jnp.sum(jnp.exp(s), axis=1, keepdims=True))
        o_ref[...] = s - lse

    return pl.pallas_call(
        body,
        out_shape=jax.ShapeDtypeStruct((n, c), jnp.float32),
    )(acc_a, acc_b, g2, dinv_col, w2, b2_row)


def kernel(x, edge_index, W1, b1, W2, b2):
    n, d = x.shape
    e = edge_index.shape[1]
    h = W1.shape[1]

    k = 80
    chunks = e // (NW * k)
    assert chunks * NW * k == e and chunks >= 2 * _NBUF
    assert n % NS == 0 and n % LANES == 0

    src3 = edge_index[0].reshape(NW, chunks, k)
    dst3 = edge_index[1].reshape(NW, chunks, k)

    zeros_rows = jnp.zeros((n, LANES), jnp.float32)

    hist = _sc_degree_hist(edge_index[1], n)
    h1 = _tc_matmul(x, W1)

    dinv8 = _tc_dinv(hist)
    dinv_col = dinv8[0].reshape(n, 1)

    g1 = _tc_scale(h1, dinv_col)
    acc1 = _sc_edge_pass(g1, src3, dst3, zeros_rows, n, chunks, k)
    g2 = _tc_layer1_post(acc1[0], acc1[1], g1, dinv_col, b1.reshape(1, h))

    acc2 = _sc_edge_pass(g2, src3, dst3, zeros_rows, n, chunks, k)
    out = _tc_layer2_post(acc2[0], acc2[1], g2, dinv_col,
                          W2, b2.reshape(1, W2.shape[1]))
    return out


# speedup vs baseline: 58.9601x; 1.1016x over previous
import dataclasses
import functools

import jax
import jax.numpy as jnp
from jax import lax
from jax.experimental import pallas as pl
from jax.experimental.pallas import tpu as pltpu
from jax.experimental.pallas import tpu_sc as plsc

NC = 2
NS = 16
NW = NC * NS
LANES = 16


def _sc_compiler_params():
    cp = pltpu.CompilerParams()
    fields = pltpu.CompilerParams.__dataclass_fields__
    if "needs_layout_passes" in fields:
        cp = dataclasses.replace(cp, needs_layout_passes=False)
    if "use_tc_tiling_on_sc" in fields:
        cp = dataclasses.replace(cp, use_tc_tiling_on_sc=False)
    return cp


def _sc_degree_hist(dst, n_nodes):
    (e_total,) = dst.shape
    epw = e_total // NW
    mesh = plsc.VectorSubcoreMesh(core_axis_name="c", subcore_axis_name="s")

    @functools.partial(
        pl.kernel,
        out_type=jax.ShapeDtypeStruct((NW, n_nodes), jnp.float32),
        mesh=mesh,
        compiler_params=_sc_compiler_params(),
        scratch_types=[
            pltpu.VMEM((epw,), jnp.int32),
            pltpu.VMEM((n_nodes,), jnp.float32),
            pltpu.SemaphoreType.DMA,
        ],
    )
    def hist_kernel(dst_hbm, out_hbm, idx_v, hist_v, sem):
        cid = lax.axis_index("c")
        sid = lax.axis_index("s")
        wid = sid * NC + cid

        zeros = jnp.zeros((LANES,), jnp.float32)

        @pl.loop(0, n_nodes // LANES)
        def _(i):
            hist_v[pl.ds(i * LANES, LANES)] = zeros

        pltpu.async_copy(dst_hbm.at[pl.ds(wid * epw, epw)], idx_v, sem).wait()

        ones = jnp.ones((LANES,), jnp.float32)

        @pl.loop(0, epw // LANES)
        def _(i):
            idx = idx_v[pl.ds(i * LANES, LANES)]
            plsc.addupdate_scatter(hist_v, [idx], ones)

        pltpu.async_copy(hist_v, out_hbm.at[wid], sem).wait()

    return hist_kernel(dst)


_NBUF = 8
_LOOK = 4


def _sc_edge_pass(g, src3, dst3, zeros_rows, n, chunks, k):
    mesh = plsc.VectorSubcoreMesh(core_axis_name="c", subcore_axis_name="s")
    rps = n // NS

    @functools.partial(
        pl.kernel,
        out_type=jax.ShapeDtypeStruct((NC, n, LANES), jnp.float32),
        mesh=mesh,
        compiler_params=_sc_compiler_params(),
        scratch_types=[
            pltpu.VMEM((chunks, k), jnp.int32),
            pltpu.VMEM((chunks, k), jnp.int32),
            pltpu.VMEM_SHARED((n, LANES), jnp.float32),
        ] + [pltpu.VMEM((k, LANES), jnp.float32)] * _NBUF
          + [pltpu.SemaphoreType.DMA] * (2 * _NBUF),
    )
    def edge_kernel(g_hbm, src_hbm, dst_hbm, z_hbm, out_hbm,
                    sidx_v, didx_v, acc_sh, *bufs_and_sems):
        rows = bufs_and_sems[:_NBUF]
        gsem = bufs_and_sems[_NBUF:2 * _NBUF]
        ssem = bufs_and_sems[2 * _NBUF:]
        cid = lax.axis_index("c")
        sid = lax.axis_index("s")
        wid = sid * NC + cid

        def issue_gather(j, b):
            pltpu.async_copy(g_hbm.at[sidx_v.at[j]], rows[b], gsem[b])

        def wait_gather(b):
            pltpu.make_async_copy(g_hbm.at[sidx_v.at[0]], rows[b],
                                  gsem[b]).wait()

        def issue_scatter(j, b):
            pltpu.async_copy(rows[b], acc_sh.at[didx_v.at[j]], ssem[b],
                             add=True)

        def wait_scatter(b):
            pltpu.make_async_copy(rows[b], acc_sh.at[didx_v.at[0]],
                                  ssem[b]).wait()

        pltpu.async_copy(
            z_hbm, acc_sh.at[pl.ds(sid * rps, rps)], gsem[0]).wait()
        pltpu.async_copy(src_hbm.at[wid], sidx_v, gsem[0]).wait()
        pltpu.async_copy(dst_hbm.at[wid], didx_v, gsem[1]).wait()
        plsc.subcore_barrier()

        for r in range(_LOOK):
            issue_gather(r, r)

        @pl.loop(0, (chunks + _NBUF - 1) // _NBUF)
        def _(q):
            for r in range(_NBUF):
                j = q * _NBUF + r
                bg = (r + _LOOK) % _NBUF

                @pl.when(j + _LOOK < chunks)
                def _():
                    @pl.when(j >= _LOOK)
                    def _():
                        wait_scatter(bg)
                    issue_gather(j + _LOOK, bg)

                @pl.when(j < chunks)
                def _():
                    wait_gather(r)
                    issue_scatter(j, r)

        for r in range(_NBUF):
            wait_scatter(r)

        plsc.subcore_barrier()
        pltpu.async_copy(
            acc_sh.at[pl.ds(sid * rps, rps)],
            out_hbm.at[cid, pl.ds(sid * rps, rps)], gsem[0]).wait()

    return edge_kernel(g, src3, dst3, zeros_rows)


def _tc_pre(hist, x, w1):
    nw, n = hist.shape
    h = w1.shape[1]

    def body(hist_ref, x_ref, w_ref, g_ref, d_ref):
        ones = jnp.ones((nw, 1), jnp.float32)
        deg = lax.dot_general(hist_ref[...], ones, (((0,), (0,)), ((), ())),
                              preferred_element_type=jnp.float32)
        dinv = lax.rsqrt(deg + 1.0)
        h1 = jnp.dot(x_ref[...], w_ref[...],
                     preferred_element_type=jnp.float32)
        d_ref[...] = dinv
        g_ref[...] = h1 * dinv

    return pl.pallas_call(
        body,
        out_shape=[
            jax.ShapeDtypeStruct((n, h), jnp.float32),
            jax.ShapeDtypeStruct((n, 1), jnp.float32),
        ],
    )(hist, x, w1)


def _tc_layer1_post(acc, g1, dinv_col, b1_row):
    def body(a_ref, g_ref, d_ref, bias_ref, o_ref):
        out1 = d_ref[...] * (a_ref[0] + a_ref[1] + g_ref[...])
        z = jnp.maximum(out1 + bias_ref[...], 0.0)
        o_ref[...] = d_ref[...] * z

    return pl.pallas_call(
        body,
        out_shape=jax.ShapeDtypeStruct(g1.shape, jnp.float32),
    )(acc, g1, dinv_col, b1_row)


def _tc_layer2_post(acc, g2, dinv_col, w2, b2_row):
    n = g2.shape[0]
    c = w2.shape[1]

    def body(a_ref, g_ref, d_ref, w_ref, bias_ref, o_ref):
        agg = d_ref[...] * (a_ref[0] + a_ref[1] + g_ref[...])
        o = jnp.dot(agg, w_ref[...], preferred_element_type=jnp.float32)
        o = o + bias_ref[...]
        m = jnp.max(o, axis=1, keepdims=True)
        s = o - m
        lse = jnp.log(jnp.sum(jnp.exp(s), axis=1, keepdims=True))
        o_ref[...] = s - lse

    return pl.pallas_call(
        body,
        out_shape=jax.ShapeDtypeStruct((n, c), jnp.float32),
    )(acc, g2, dinv_col, w2, b2_row)


def kernel(x, edge_index, W1, b1, W2, b2):
    n, d = x.shape
    e = edge_index.shape[1]
    h = W1.shape[1]

    k = 80
    chunks = e // (NW * k)
    assert chunks * NW * k == e and chunks >= 2 * _NBUF
    assert n % NS == 0 and n % LANES == 0

    src3 = edge_index[0].reshape(NW, chunks, k)
    dst3 = edge_index[1].reshape(NW, chunks, k)

    zeros_stripe = jnp.zeros((n // NS, LANES), jnp.float32)

    hist = _sc_degree_hist(edge_index[1], n)
    g1, dinv_col = _tc_pre(hist, x, W1)

    acc1 = _sc_edge_pass(g1, src3, dst3, zeros_stripe, n, chunks, k)
    g2 = _tc_layer1_post(acc1, g1, dinv_col, b1.reshape(1, h))

    acc2 = _sc_edge_pass(g2, src3, dst3, zeros_stripe, n, chunks, k)
    out = _tc_layer2_post(acc2, g2, dinv_col, W2, b2.reshape(1, W2.shape[1]))
    return out
